# Initial kernel scaffold; baseline (speedup 1.0000x reference)
import jax
import jax.numpy as jnp
from jax.experimental import pallas as pl


def kernel(input_ids):
    raise NotImplementedError("write your pallas kernel here")


# speedup vs baseline: 1.0307x; 1.0307x over previous
import jax
import jax.numpy as jnp
from jax.experimental import pallas as pl
import numpy as np

_ROT_A = (13, 15, 26, 6)
_ROT_B = (17, 29, 16, 24)
_M32 = 0xFFFFFFFF


def _threefry2x32_scalar(k0, k1, x0, x1):
    ks = (k0, k1, (0x1BD11BDA ^ k0 ^ k1) & _M32)
    x0 = (x0 + k0) & _M32
    x1 = (x1 + k1) & _M32
    for i, rots in enumerate((_ROT_A, _ROT_B, _ROT_A, _ROT_B, _ROT_A)):
        for r in rots:
            x0 = (x0 + x1) & _M32
            x1 = ((x1 << r) | (x1 >> (32 - r))) & _M32
            x1 ^= x0
        x0 = (x0 + ks[(i + 1) % 3]) & _M32
        x1 = (x1 + ks[(i + 2) % 3] + i + 1) & _M32
    return x0, x1


_K0, _K1 = _threefry2x32_scalar(0, 0, 0, 1234)
_KS2 = (0x1BD11BDA ^ _K0 ^ _K1) & _M32
_THRESH = 1677721

_ROWS, _COLS = 128, 8192
_BLK_ROWS = 16


def _body(x_ref, o_ref):
    r0 = pl.program_id(0) * _BLK_ROWS
    row = jax.lax.broadcasted_iota(jnp.uint32, (_BLK_ROWS, _COLS), 0)
    col = jax.lax.broadcasted_iota(jnp.uint32, (_BLK_ROWS, _COLS), 1)
    idx = (jnp.uint32(r0) + row) * jnp.uint32(_COLS) + col

    ks = (jnp.uint32(_K0), jnp.uint32(_K1), jnp.uint32(_KS2))
    x0 = jnp.full((_BLK_ROWS, _COLS), _K0, jnp.uint32)
    x1 = idx + jnp.uint32(_K1)
    for i, rots in enumerate((_ROT_A, _ROT_B, _ROT_A, _ROT_B, _ROT_A)):
        for r in rots:
            x0 = x0 + x1
            x1 = (x1 << jnp.uint32(r)) | (x1 >> jnp.uint32(32 - r))
            x1 = x1 ^ x0
        x0 = x0 + ks[(i + 1) % 3]
        x1 = x1 + ks[(i + 2) % 3] + jnp.uint32(i + 1)

    bits = x0 ^ x1
    x = x_ref[...]
    drop = ((bits >> jnp.uint32(9)) <= jnp.uint32(_THRESH)) & (x != 0) & (x != 2)
    o_ref[...] = jnp.where(drop, jnp.zeros_like(x), x)


def kernel(input_ids):
    return pl.pallas_call(
        _body,
        grid=(_ROWS // _BLK_ROWS,),
        in_specs=[pl.BlockSpec((_BLK_ROWS, _COLS), lambda i: (i, 0))],
        out_specs=pl.BlockSpec((_BLK_ROWS, _COLS), lambda i: (i, 0)),
        out_shape=jax.ShapeDtypeStruct(input_ids.shape, input_ids.dtype),
    )(input_ids)


# speedup vs baseline: 1.0657x; 1.0339x over previous
import jax
import jax.numpy as jnp
from jax.experimental import pallas as pl
import numpy as np

_ROT_A = (13, 15, 26, 6)
_ROT_B = (17, 29, 16, 24)
_M32 = 0xFFFFFFFF


def _threefry2x32_scalar(k0, k1, x0, x1):
    ks = (k0, k1, (0x1BD11BDA ^ k0 ^ k1) & _M32)
    x0 = (x0 + k0) & _M32
    x1 = (x1 + k1) & _M32
    for i, rots in enumerate((_ROT_A, _ROT_B, _ROT_A, _ROT_B, _ROT_A)):
        for r in rots:
            x0 = (x0 + x1) & _M32
            x1 = ((x1 << r) | (x1 >> (32 - r))) & _M32
            x1 ^= x0
        x0 = (x0 + ks[(i + 1) % 3]) & _M32
        x1 = (x1 + ks[(i + 2) % 3] + i + 1) & _M32
    return x0, x1


_K0, _K1 = _threefry2x32_scalar(0, 0, 0, 1234)
_KS2 = (0x1BD11BDA ^ _K0 ^ _K1) & _M32
_THRESH = 1677721

_ROWS, _COLS = 128, 8192
_BLK_ROWS = 16


def _body(x_ref, o_ref):
    r0 = pl.program_id(0) * _BLK_ROWS
    row = jax.lax.broadcasted_iota(jnp.uint32, (_BLK_ROWS, _COLS), 0)
    col = jax.lax.broadcasted_iota(jnp.uint32, (_BLK_ROWS, _COLS), 1)

    ks = (_K0, _K1, _KS2)
    base = jnp.uint32(r0) * jnp.uint32(_COLS) + jnp.uint32(_K1)
    x1 = (row << jnp.uint32(13)) + col + base
    rounds = []
    for i, rots in enumerate((_ROT_A, _ROT_B, _ROT_A, _ROT_B, _ROT_A)):
        c0 = ks[(i + 1) % 3]
        c1 = (ks[(i + 2) % 3] + i + 1) & _M32
        rounds.append((rots, jnp.uint32(c0), jnp.uint32(c1)))

    x0 = x1 + jnp.uint32(_K0)
    first = True
    for rots, c0, c1 in rounds:
        for r in rots:
            if first:
                first = False
            else:
                x0 = x0 + x1
            x1 = (x1 << jnp.uint32(r)) | (x1 >> jnp.uint32(32 - r))
            x1 = x1 ^ x0
        x0 = x0 + c0
        x1 = x1 + c1

    bits = x0 ^ x1
    x = x_ref[...]
    drop = ((bits >> jnp.uint32(9)) <= jnp.uint32(_THRESH)) & (x != 0) & (x != 2)
    o_ref[...] = jnp.where(drop, jnp.zeros_like(x), x)


def kernel(input_ids):
    return pl.pallas_call(
        _body,
        grid=(_ROWS // _BLK_ROWS,),
        in_specs=[pl.BlockSpec((_BLK_ROWS, _COLS), lambda i: (i, 0))],
        out_specs=pl.BlockSpec((_BLK_ROWS, _COLS), lambda i: (i, 0)),
        out_shape=jax.ShapeDtypeStruct(input_ids.shape, input_ids.dtype),
    )(input_ids)
